# baseline (speedup 1.0000x reference)
import functools

import jax
import jax.numpy as jnp
from jax import lax
from jax.experimental import pallas as pl
from jax.experimental.pallas import tpu as pltpu
from jax.experimental.pallas import tpu_sc as plsc

BATCH = 16384
D_EMB = 64
D_FEAT = 128
N_USERS = 1000000
N_ITEMS = 100000
NC = 2
NS = 16
NW = NC * NS
B_PER_W = BATCH // NW
CHF = 128
NCHF = B_PER_W // CHF
L = 16


@functools.cache
def _build_sc_gather():
    mesh = plsc.VectorSubcoreMesh(core_axis_name="c", subcore_axis_name="s")

    @functools.partial(
        pl.kernel,
        out_type=(
            jax.ShapeDtypeStruct((2 * D_EMB, BATCH), jnp.float32),
            jax.ShapeDtypeStruct((BATCH, D_FEAT), jnp.float32),
            jax.ShapeDtypeStruct((BATCH, D_FEAT), jnp.float32),
        ),
        mesh=mesh,
        compiler_params=pltpu.CompilerParams(use_tc_tiling_on_sc=False,
                                             needs_layout_passes=False),
        scratch_types=[
            pltpu.VMEM((B_PER_W,), jnp.int32),
            pltpu.VMEM((B_PER_W,), jnp.int32),
            pltpu.VMEM((D_EMB, 8 * L), jnp.float32),
            pltpu.VMEM((D_EMB, 8 * L), jnp.float32),
            pltpu.VMEM((2 * D_EMB, B_PER_W), jnp.float32),
            pltpu.VMEM((CHF,), jnp.int32),
            pltpu.VMEM((CHF, D_FEAT), jnp.float32),
            pltpu.VMEM((CHF, D_FEAT), jnp.float32),
            pltpu.SemaphoreType.DMA,
        ],
    )
    def _sc_gather(uid_hbm, iid_hbm, utT_hbm, itT_hbm, vf_hbm, tf_hbm,
                   ui_out, v_out, t_out,
                   idu, idi, ubuf, ibuf, pkT, fidx, vbuf, tbuf, sem):
        wid = lax.axis_index("s") * NC + lax.axis_index("c")
        base = wid * B_PER_W
        iota = lax.iota(jnp.int32, L)

        pltpu.sync_copy(uid_hbm.at[pl.ds(base, B_PER_W)], idu)
        pltpu.sync_copy(iid_hbm.at[pl.ds(base, B_PER_W)], idi)

        def grp(g, carry):
            ids_u = idu[pl.ds(g * L, L)]
            ids_i = idi[pl.ds(g * L, L)]
            for l in range(L):
                pltpu.make_async_copy(
                    utT_hbm.at[pl.ds(0, D_EMB),
                               pl.ds((ids_u[l] // 8) * 8, 8)],
                    ubuf.at[pl.ds(0, D_EMB), pl.ds(l * 8, 8)], sem).start()
                pltpu.make_async_copy(
                    itT_hbm.at[pl.ds(0, D_EMB),
                               pl.ds((ids_i[l] // 8) * 8, 8)],
                    ibuf.at[pl.ds(0, D_EMB), pl.ds(l * 8, 8)], sem).start()
            pltpu.make_async_copy(
                utT_hbm.at[pl.ds(0, D_EMB), pl.ds(0, 8 * L)], ubuf,
                sem).wait()
            pltpu.make_async_copy(
                itT_hbm.at[pl.ds(0, D_EMB), pl.ds(0, 8 * L)], ibuf,
                sem).wait()
            colu = iota * 8 + lax.rem(ids_u, 8)
            coli = iota * 8 + lax.rem(ids_i, 8)
            for c in range(D_EMB):
                cs = jnp.full((L,), c, jnp.int32)
                pkT[c, pl.ds(g * L, L)] = plsc.load_gather(ubuf, [cs, colu])
                pkT[D_EMB + c, pl.ds(g * L, L)] = plsc.load_gather(
                    ibuf, [cs, coli])
            return carry

        lax.fori_loop(0, B_PER_W // L, grp, 0)
        pltpu.sync_copy(pkT, ui_out.at[pl.ds(0, 2 * D_EMB),
                                       pl.ds(base, B_PER_W)])

        for c in range(NCHF):
            off = base + c * CHF
            pltpu.sync_copy(iid_hbm.at[pl.ds(off, CHF)], fidx)
            gv = pltpu.async_copy(vf_hbm.at[fidx], vbuf, sem)
            gt = pltpu.async_copy(tf_hbm.at[fidx], tbuf, sem)
            gv.wait()
            gt.wait()
            pltpu.sync_copy(vbuf, v_out.at[pl.ds(off, CHF)])
            pltpu.sync_copy(tbuf, t_out.at[pl.ds(off, CHF)])


    return _sc_gather


BLK = 1024


def _tc_body(ui_ref, v_ref, t_ref, wv_ref, wt_ref, wf_ref, bf_ref,
             out_ref):
    wf = wf_ref[...]
    a_vis = lax.dot_general(wf[:, :D_EMB], wv_ref[...],
                            (((1,), (0,)), ((), ())),
                            preferred_element_type=jnp.float32)
    a_txt = lax.dot_general(wf[:, D_EMB:], wt_ref[...],
                            (((1,), (0,)), ((), ())),
                            preferred_element_type=jnp.float32)
    mm_t = lax.dot_general(a_vis, v_ref[...], (((1,), (1,)), ((), ())),
                           preferred_element_type=jnp.float32)
    mm_t = mm_t + lax.dot_general(a_txt, t_ref[...], (((1,), (1,)), ((), ())),
                                  preferred_element_type=jnp.float32)
    mm_t = mm_t + bf_ref[...]
    ui = ui_ref[...]
    u_t = ui[:D_EMB, :]
    i_t = ui[D_EMB:, :]
    out_ref[...] = jnp.sum(u_t * (i_t + mm_t), axis=0)


_tc_score = pl.pallas_call(
    _tc_body,
    grid=(BATCH // BLK,),
    in_specs=[
        pl.BlockSpec((2 * D_EMB, BLK), lambda i: (0, i)),
        pl.BlockSpec((BLK, D_FEAT), lambda i: (i, 0)),
        pl.BlockSpec((BLK, D_FEAT), lambda i: (i, 0)),
        pl.BlockSpec((D_EMB, D_FEAT), lambda i: (0, 0)),
        pl.BlockSpec((D_EMB, D_FEAT), lambda i: (0, 0)),
        pl.BlockSpec((D_EMB, 2 * D_EMB), lambda i: (0, 0)),
        pl.BlockSpec((D_EMB, 1), lambda i: (0, 0)),
    ],
    out_specs=pl.BlockSpec((BLK,), lambda i: (i,)),
    out_shape=jax.ShapeDtypeStruct((BATCH,), jnp.float32),
)


def kernel(user_ids, item_ids, user_table, item_table, visual_features,
           text_features, W_vis, W_txt, W_fuse, b_fuse):
    uid = user_ids.astype(jnp.int32)
    iid = item_ids.astype(jnp.int32)
    ui_g, v_g, t_g = _build_sc_gather()(uid, iid, user_table.T, item_table.T,
                                        visual_features, text_features)
    return _tc_score(ui_g, v_g, t_g, W_vis, W_txt, W_fuse,
                     b_fuse.reshape(D_EMB, 1))


# speedup vs baseline: 12.0840x; 12.0840x over previous
import functools

import jax
import jax.numpy as jnp
from jax import lax
from jax.experimental import pallas as pl
from jax.experimental.pallas import tpu as pltpu
from jax.experimental.pallas import tpu_sc as plsc

BATCH = 16384
D_EMB = 64
D_FEAT = 128
N_USERS = 1000000
N_ITEMS = 100000
NC = 2
NS = 16
NW = NC * NS
B_PER_W = BATCH // NW
CHF = 128
NCHF = B_PER_W // CHF
L = 16


@functools.cache
def _build_sc_gather():
    mesh = plsc.VectorSubcoreMesh(core_axis_name="c", subcore_axis_name="s")

    @functools.partial(
        pl.kernel,
        out_type=(
            jax.ShapeDtypeStruct((BATCH, D_EMB), jnp.float32),
            jax.ShapeDtypeStruct((BATCH, D_EMB), jnp.float32),
            jax.ShapeDtypeStruct((BATCH, D_FEAT), jnp.float32),
            jax.ShapeDtypeStruct((BATCH, D_FEAT), jnp.float32),
        ),
        mesh=mesh,
        scratch_types=[
            pltpu.VMEM((B_PER_W,), jnp.int32),
            pltpu.VMEM((B_PER_W, D_EMB), jnp.float32),
            pltpu.VMEM((CHF,), jnp.int32),
            pltpu.VMEM((CHF, D_FEAT), jnp.float32),
            pltpu.VMEM((CHF, D_FEAT), jnp.float32),
            pltpu.SemaphoreType.DMA,
        ],
    )
    def _sc_gather(uid_hbm, iid_hbm, ut_hbm, it_hbm, vf_hbm, tf_hbm,
                   u_out, i_out, v_out, t_out,
                   idx_v, rows, fidx, vbuf, tbuf, sem):
        wid = lax.axis_index("s") * NC + lax.axis_index("c")
        base = wid * B_PER_W

        def gather_64wide(ids_hbm, tab_hbm, out_hbm):
            pltpu.sync_copy(ids_hbm.at[pl.ds(base, B_PER_W)], idx_v)

            def fire_group(g, carry):
                ids16 = idx_v[pl.ds(g * L, L)]
                for k in range(L):
                    pltpu.make_async_copy(
                        tab_hbm.at[pl.ds(ids16[k], 1)],
                        rows.at[pl.ds(g * L + k, 1)], sem).start()
                return carry

            lax.fori_loop(0, B_PER_W // L, fire_group, 0)
            pltpu.make_async_copy(
                tab_hbm.at[pl.ds(0, B_PER_W)], rows, sem).wait()
            pltpu.sync_copy(rows, out_hbm.at[pl.ds(base, B_PER_W)])

        gather_64wide(uid_hbm, ut_hbm, u_out)
        gather_64wide(iid_hbm, it_hbm, i_out)

        for c in range(NCHF):
            off = base + c * CHF
            pltpu.sync_copy(iid_hbm.at[pl.ds(off, CHF)], fidx)
            gv = pltpu.async_copy(vf_hbm.at[fidx], vbuf, sem)
            gt = pltpu.async_copy(tf_hbm.at[fidx], tbuf, sem)
            gv.wait()
            gt.wait()
            pltpu.sync_copy(vbuf, v_out.at[pl.ds(off, CHF)])
            pltpu.sync_copy(tbuf, t_out.at[pl.ds(off, CHF)])

    return _sc_gather


BLK = 1024


def _tc_body(u_ref, i_ref, v_ref, t_ref, wv_ref, wt_ref, wf_ref, bf_ref,
             out_ref):
    wf = wf_ref[...]
    a_vis = lax.dot_general(wf[:, :D_EMB], wv_ref[...],
                            (((1,), (0,)), ((), ())),
                            preferred_element_type=jnp.float32)
    a_txt = lax.dot_general(wf[:, D_EMB:], wt_ref[...],
                            (((1,), (0,)), ((), ())),
                            preferred_element_type=jnp.float32)
    mm = lax.dot_general(v_ref[...], a_vis, (((1,), (1,)), ((), ())),
                         preferred_element_type=jnp.float32)
    mm = mm + lax.dot_general(t_ref[...], a_txt, (((1,), (1,)), ((), ())),
                              preferred_element_type=jnp.float32)
    mm = mm + bf_ref[...]
    out_ref[...] = jnp.sum(u_ref[...] * (i_ref[...] + mm), axis=1)


_tc_score = pl.pallas_call(
    _tc_body,
    grid=(BATCH // BLK,),
    in_specs=[
        pl.BlockSpec((BLK, D_EMB), lambda i: (i, 0)),
        pl.BlockSpec((BLK, D_EMB), lambda i: (i, 0)),
        pl.BlockSpec((BLK, D_FEAT), lambda i: (i, 0)),
        pl.BlockSpec((BLK, D_FEAT), lambda i: (i, 0)),
        pl.BlockSpec((D_EMB, D_FEAT), lambda i: (0, 0)),
        pl.BlockSpec((D_EMB, D_FEAT), lambda i: (0, 0)),
        pl.BlockSpec((D_EMB, 2 * D_EMB), lambda i: (0, 0)),
        pl.BlockSpec((1, D_EMB), lambda i: (0, 0)),
    ],
    out_specs=pl.BlockSpec((BLK,), lambda i: (i,)),
    out_shape=jax.ShapeDtypeStruct((BATCH,), jnp.float32),
)


def kernel(user_ids, item_ids, user_table, item_table, visual_features,
           text_features, W_vis, W_txt, W_fuse, b_fuse):
    uid = user_ids.astype(jnp.int32)
    iid = item_ids.astype(jnp.int32)
    u_g, i_g, v_g, t_g = _build_sc_gather()(uid, iid, user_table, item_table,
                                            visual_features, text_features)
    return _tc_score(u_g, i_g, v_g, t_g, W_vis, W_txt, W_fuse,
                     b_fuse.reshape(1, D_EMB))


# speedup vs baseline: 18.0855x; 1.4966x over previous
import functools

import jax
import jax.numpy as jnp
from jax import lax
from jax.experimental import pallas as pl
from jax.experimental.pallas import tpu as pltpu
from jax.experimental.pallas import tpu_sc as plsc

BATCH = 16384
D_EMB = 64
D_FEAT = 128
N_USERS = 1000000
N_ITEMS = 100000
NC = 2
NS = 16
NW = NC * NS
B_PER_W = BATCH // NW
CHF = 64
NCHF = B_PER_W // CHF
L = 16
WBLK = 4
MB = 2048
OUT_ROWS = BATCH + 1024


@functools.cache
def _build_sc_gather():
    mesh = plsc.VectorSubcoreMesh(core_axis_name="c", subcore_axis_name="s")

    @functools.partial(
        pl.kernel,
        out_type=(
            jax.ShapeDtypeStruct((OUT_ROWS, D_FEAT), jnp.float32),
            jax.ShapeDtypeStruct((OUT_ROWS, D_FEAT), jnp.float32),
            jax.ShapeDtypeStruct((BATCH, D_FEAT), jnp.float32),
            jax.ShapeDtypeStruct((BATCH, D_FEAT), jnp.float32),
        ),
        mesh=mesh,
        compiler_params=pltpu.CompilerParams(use_tc_tiling_on_sc=True,
                                             needs_layout_passes=False),
        scratch_types=[
            pltpu.VMEM((BATCH,), jnp.int32),
            pltpu.VMEM((MB + L,), jnp.int32),
            pltpu.VMEM((MB + L,), jnp.int32),
            pltpu.VMEM((MB + L,), jnp.int32),
            pltpu.VMEM((MB + L,), jnp.int32),
            pltpu.VMEM((2, WBLK, D_EMB, 128), jnp.float32),
            pltpu.VMEM((2, L, D_FEAT), jnp.float32),
            pltpu.VMEM((D_EMB, D_EMB), jnp.float32),
            pltpu.VMEM((CHF,), jnp.int32),
            pltpu.VMEM((CHF, D_FEAT), jnp.float32),
            pltpu.VMEM((CHF, D_FEAT), jnp.float32),
            pltpu.SemaphoreType.DMA,
            pltpu.SemaphoreType.DMA,
        ],
    )
    def _sc_gather(uid_hbm, iid_hbm, utT_hbm, itT_hbm, tlu_hbm, tli_hbm,
                   vf_hbm, tf_hbm,
                   u_out, i_out, v_out, t_out,
                   ids, ml, mb, su, sb, ring, stage, tl, fidx, vbuf, tbuf,
                   sem, sem2):
        wid = lax.axis_index("s") * NC + lax.axis_index("c")
        base = wid * B_PER_W
        iota = lax.iota(jnp.int32, L)
        dump = BATCH + wid

        def pass64(ids_hbm, tabT_hbm, tail_hbm, out_hbm, n_rows):
            rng = n_rows // NW
            lo = wid * rng
            hi = lo + rng
            kb0 = lo // 128
            kb1 = (hi - 1) // 128
            nwin = (kb1 - kb0) // WBLK + 1
            nfull = n_rows // 128
            tailw = n_rows - nfull * 128

            pltpu.sync_copy(ids_hbm, ids)
            pltpu.sync_copy(tail_hbm, tl.at[pl.ds(0, tailw)])

            def count_body(s, cnt):
                v = ids[pl.ds(s * L, L)]
                m = (v >= lo) & (v < hi)
                return cnt + plsc.all_reduce_population_count(m)[0]

            total = lax.fori_loop(0, BATCH // L, count_body, 0)
            nrounds = lax.div(total + MB - 1, MB)

            def fetch_win(w, half, start):
                for q in range(WBLK):
                    blk = kb0 + w * WBLK + q

                    @pl.when((blk <= kb1) & (blk < nfull))
                    def _():
                        cp = pltpu.make_async_copy(
                            tabT_hbm.at[pl.ds(0, D_EMB),
                                        pl.ds(pl.multiple_of(blk * 128, 128),
                                              128)],
                            ring.at[half, q], sem)
                        if start:
                            cp.start()
                        else:
                            cp.wait()

            def round_body(r, carry0):
                def comp_body(s, carry):
                    off, grc = carry
                    v = ids[pl.ds(s * L, L)]
                    m = (v >= lo) & (v < hi)
                    mi = m.astype(jnp.int32)
                    rank = grc + plsc.cumsum(mi) - 1
                    sel = m & (rank >= r * MB) & (rank < r * MB + MB)
                    plsc.store_compressed(ml.at[pl.ds(off, L)], v, mask=sel)
                    plsc.store_compressed(mb.at[pl.ds(off, L)],
                                          iota + s * L, mask=sel)
                    noff = off + plsc.all_reduce_population_count(sel)[0]
                    ngrc = grc + plsc.all_reduce_population_count(m)[0]
                    return (noff, ngrc)

                mcnt, _ = lax.fori_loop(0, BATCH // L, comp_body, (0, 0))

                fetch_win(0, 0, True)

                def win_body(w, carry1):
                    half = lax.rem(w, 2)
                    fetch_win(w, half, False)

                    @pl.when(w + 1 < nwin)
                    def _():
                        fetch_win(w + 1, 1 - half, True)

                    wb0 = kb0 + w * WBLK

                    def sub_body(s, scnt):
                        v = ml[pl.ds(s * L, L)]
                        b = mb[pl.ds(s * L, L)]
                        valid = (iota + s * L) < mcnt
                        blkv = lax.shift_right_logical(v, 7)
                        m = valid & (blkv >= wb0) & (blkv < wb0 + WBLK)
                        plsc.store_compressed(su.at[pl.ds(scnt, L)], v, mask=m)
                        plsc.store_compressed(sb.at[pl.ds(scnt, L)], b, mask=m)
                        return scnt + plsc.all_reduce_population_count(m)[0]

                    scnt = lax.fori_loop(0, lax.div(mcnt + L - 1, L),
                                         sub_body, 0)
                    su[pl.ds(scnt, L)] = jnp.full((L,), 0, jnp.int32) + \
                        wb0 * 128
                    sb[pl.ds(scnt, L)] = jnp.full((L,), 0, jnp.int32) + dump

                    def grp_body(g, carry2):
                        gh = lax.rem(g, 2)
                        v = su[pl.ds(g * L, L)]
                        b = sb[pl.ds(g * L, L)]
                        blkv = lax.shift_right_logical(v, 7)
                        qv = jnp.minimum(blkv - wb0, WBLK - 1)
                        colv = lax.rem(v, 128)
                        is_tail = blkv >= nfull
                        ov = jnp.clip(v - nfull * 128, 0, tailw - 1)
                        for c in range(D_EMB):
                            cs = jnp.full((L,), c, jnp.int32)
                            val = plsc.load_gather(ring.at[half],
                                                   [qv, cs, colv])
                            tval = plsc.load_gather(tl, [ov, cs])
                            val = jnp.where(is_tail, tval, val)
                            plsc.store_scatter(stage.at[gh], [iota, cs], val)
                        pltpu.async_copy(stage.at[gh], out_hbm.at[b],
                                         sem2).wait()
                        return carry2

                    lax.fori_loop(0, lax.div(scnt + L - 1, L), grp_body, 0)
                    return carry1

                lax.fori_loop(0, nwin, win_body, 0)
                return carry0

            lax.fori_loop(0, nrounds, round_body, 0)

        pass64(uid_hbm, utT_hbm, tlu_hbm, u_out, N_USERS)
        pass64(iid_hbm, itT_hbm, tli_hbm, i_out, N_ITEMS)

        for c in range(NCHF):
            off = base + c * CHF
            pltpu.sync_copy(iid_hbm.at[pl.ds(off, CHF)], fidx)
            gv = pltpu.async_copy(vf_hbm.at[fidx], vbuf, sem)
            gt = pltpu.async_copy(tf_hbm.at[fidx], tbuf, sem)
            gv.wait()
            gt.wait()
            pltpu.sync_copy(vbuf, v_out.at[pl.ds(off, CHF)])
            pltpu.sync_copy(tbuf, t_out.at[pl.ds(off, CHF)])

    return _sc_gather


BLK = 1024


def _tc_body(u_ref, i_ref, v_ref, t_ref, wv_ref, wt_ref, wf_ref, bf_ref,
             out_ref):
    wf = wf_ref[...]
    a_vis = lax.dot_general(wf[:, :D_EMB], wv_ref[...],
                            (((1,), (0,)), ((), ())),
                            preferred_element_type=jnp.float32)
    a_txt = lax.dot_general(wf[:, D_EMB:], wt_ref[...],
                            (((1,), (0,)), ((), ())),
                            preferred_element_type=jnp.float32)
    mm = lax.dot_general(v_ref[...], a_vis, (((1,), (1,)), ((), ())),
                         preferred_element_type=jnp.float32)
    mm = mm + lax.dot_general(t_ref[...], a_txt, (((1,), (1,)), ((), ())),
                              preferred_element_type=jnp.float32)
    mm = mm + bf_ref[...]
    u = u_ref[...][:, :D_EMB]
    i = i_ref[...][:, :D_EMB]
    out_ref[...] = jnp.sum(u * (i + mm), axis=1)


_tc_score = pl.pallas_call(
    _tc_body,
    grid=(BATCH // BLK,),
    in_specs=[
        pl.BlockSpec((BLK, D_FEAT), lambda i: (i, 0)),
        pl.BlockSpec((BLK, D_FEAT), lambda i: (i, 0)),
        pl.BlockSpec((BLK, D_FEAT), lambda i: (i, 0)),
        pl.BlockSpec((BLK, D_FEAT), lambda i: (i, 0)),
        pl.BlockSpec((D_EMB, D_FEAT), lambda i: (0, 0)),
        pl.BlockSpec((D_EMB, D_FEAT), lambda i: (0, 0)),
        pl.BlockSpec((D_EMB, 2 * D_EMB), lambda i: (0, 0)),
        pl.BlockSpec((1, D_EMB), lambda i: (0, 0)),
    ],
    out_specs=pl.BlockSpec((BLK,), lambda i: (i,)),
    out_shape=jax.ShapeDtypeStruct((BATCH,), jnp.float32),
)


def kernel(user_ids, item_ids, user_table, item_table, visual_features,
           text_features, W_vis, W_txt, W_fuse, b_fuse):
    uid = user_ids.astype(jnp.int32)
    iid = item_ids.astype(jnp.int32)
    tlu = user_table[(N_USERS // 128) * 128:]
    tli = item_table[(N_ITEMS // 128) * 128:]
    u_g, i_g, v_g, t_g = _build_sc_gather()(uid, iid, user_table.T,
                                            item_table.T, tlu, tli,
                                            visual_features, text_features)
    return _tc_score(u_g, i_g, v_g, t_g, W_vis, W_txt, W_fuse,
                     b_fuse.reshape(1, D_EMB))


# speedup vs baseline: 18.1908x; 1.0058x over previous
import functools

import jax
import jax.numpy as jnp
from jax import lax
from jax.experimental import pallas as pl
from jax.experimental.pallas import tpu as pltpu
from jax.experimental.pallas import tpu_sc as plsc

BATCH = 16384
D_EMB = 64
D_FEAT = 128
N_USERS = 1000000
N_ITEMS = 100000
NC = 2
NS = 16
NW = NC * NS
B_PER_W = BATCH // NW
CHF = 64
NCHF = B_PER_W // CHF
L = 16
WBLK = 4
MB = 2048
OUT_ROWS = BATCH + 1024


@functools.cache
def _build_sc_gather():
    mesh = plsc.VectorSubcoreMesh(core_axis_name="c", subcore_axis_name="s")

    @functools.partial(
        pl.kernel,
        out_type=(
            jax.ShapeDtypeStruct((OUT_ROWS, D_FEAT), jnp.float32),
            jax.ShapeDtypeStruct((OUT_ROWS, D_FEAT), jnp.float32),
            jax.ShapeDtypeStruct((BATCH, D_FEAT), jnp.float32),
            jax.ShapeDtypeStruct((BATCH, D_FEAT), jnp.float32),
        ),
        mesh=mesh,
        compiler_params=pltpu.CompilerParams(use_tc_tiling_on_sc=True,
                                             needs_layout_passes=False),
        scratch_types=[
            pltpu.VMEM((BATCH,), jnp.int32),
            pltpu.VMEM((MB + L,), jnp.int32),
            pltpu.VMEM((MB + L,), jnp.int32),
            pltpu.VMEM((MB + L,), jnp.int32),
            pltpu.VMEM((MB + L,), jnp.int32),
            pltpu.VMEM((2, WBLK, D_EMB, 128), jnp.float32),
            pltpu.VMEM((2, L, D_FEAT), jnp.float32),
            pltpu.VMEM((D_EMB, D_EMB), jnp.float32),
            pltpu.VMEM((CHF,), jnp.int32),
            pltpu.VMEM((CHF, D_FEAT), jnp.float32),
            pltpu.VMEM((CHF, D_FEAT), jnp.float32),
            pltpu.SemaphoreType.DMA,
            pltpu.SemaphoreType.DMA,
        ],
    )
    def _sc_gather(uid_hbm, iid_hbm, utT_hbm, itT_hbm, tlu_hbm, tli_hbm,
                   vf_hbm, tf_hbm,
                   u_out, i_out, v_out, t_out,
                   ids, ml, mb, su, sb, ring, stage, tl, fidx, vbuf, tbuf,
                   sem, sem2):
        wid = lax.axis_index("s") * NC + lax.axis_index("c")
        base = wid * B_PER_W
        iota = lax.iota(jnp.int32, L)
        dump = BATCH + wid

        def pass64(ids_hbm, tabT_hbm, tail_hbm, out_hbm, n_rows):
            rng = n_rows // NW
            lo = wid * rng
            hi = lo + rng
            kb0 = lo // 128
            kb1 = (hi - 1) // 128
            nwin = (kb1 - kb0) // WBLK + 1
            nfull = n_rows // 128
            tailw = n_rows - nfull * 128

            pltpu.sync_copy(ids_hbm, ids)
            pltpu.sync_copy(tail_hbm, tl.at[pl.ds(0, tailw)])

            def count_body(s, cnt):
                v = ids[pl.ds(s * L, L)]
                m = (v >= lo) & (v < hi)
                return cnt + plsc.all_reduce_population_count(m)[0]

            total = lax.fori_loop(0, BATCH // L, count_body, 0)
            nrounds = lax.div(total + MB - 1, MB)

            def fetch_win(w, half, start):
                for q in range(WBLK):
                    blk = kb0 + w * WBLK + q

                    @pl.when((blk <= kb1) & (blk < nfull))
                    def _():
                        cp = pltpu.make_async_copy(
                            tabT_hbm.at[pl.ds(0, D_EMB),
                                        pl.ds(pl.multiple_of(blk * 128, 128),
                                              128)],
                            ring.at[half, q], sem)
                        if start:
                            cp.start()
                        else:
                            cp.wait()

            def round_body(r, carry0):
                def comp_body(s, carry):
                    off, grc = carry
                    v = ids[pl.ds(s * L, L)]
                    m = (v >= lo) & (v < hi)
                    mi = m.astype(jnp.int32)
                    rank = grc + plsc.cumsum(mi) - 1
                    sel = m & (rank >= r * MB) & (rank < r * MB + MB)
                    plsc.store_compressed(ml.at[pl.ds(off, L)], v, mask=sel)
                    plsc.store_compressed(mb.at[pl.ds(off, L)],
                                          iota + s * L, mask=sel)
                    noff = off + plsc.all_reduce_population_count(sel)[0]
                    ngrc = grc + plsc.all_reduce_population_count(m)[0]
                    return (noff, ngrc)

                mcnt, _ = lax.fori_loop(0, BATCH // L, comp_body, (0, 0))

                fetch_win(0, 0, True)

                def win_body(w, carry1):
                    half = lax.rem(w, 2)
                    fetch_win(w, half, False)

                    @pl.when(w + 1 < nwin)
                    def _():
                        fetch_win(w + 1, 1 - half, True)

                    wb0 = kb0 + w * WBLK

                    def sub_body(s, scnt):
                        v = ml[pl.ds(s * L, L)]
                        b = mb[pl.ds(s * L, L)]
                        valid = (iota + s * L) < mcnt
                        blkv = lax.shift_right_logical(v, 7)
                        m = valid & (blkv >= wb0) & (blkv < wb0 + WBLK)
                        plsc.store_compressed(su.at[pl.ds(scnt, L)], v, mask=m)
                        plsc.store_compressed(sb.at[pl.ds(scnt, L)], b, mask=m)
                        return scnt + plsc.all_reduce_population_count(m)[0]

                    scnt = lax.fori_loop(0, lax.div(mcnt + L - 1, L),
                                         sub_body, 0)
                    su[pl.ds(scnt, L)] = jnp.full((L,), 0, jnp.int32) + \
                        wb0 * 128
                    sb[pl.ds(scnt, L)] = jnp.full((L,), 0, jnp.int32) + dump

                    def grp_body(g, gtot):
                        gh = lax.rem(gtot, 2)

                        @pl.when(gtot >= 1)
                        def _():
                            pltpu.make_async_copy(
                                out_hbm.at[pl.ds(0, L)], stage.at[gh],
                                sem2).wait()

                        v = su[pl.ds(g * L, L)]
                        b = sb[pl.ds(g * L, L)]
                        blkv = lax.shift_right_logical(v, 7)
                        qv = jnp.minimum(blkv - wb0, WBLK - 1)
                        colv = lax.rem(v, 128)
                        is_tail = blkv >= nfull
                        ov = jnp.clip(v - nfull * 128, 0, tailw - 1)
                        for c in range(D_EMB):
                            cs = jnp.full((L,), c, jnp.int32)
                            val = plsc.load_gather(ring.at[half],
                                                   [qv, cs, colv])
                            tval = plsc.load_gather(tl, [ov, cs])
                            val = jnp.where(is_tail, tval, val)
                            plsc.store_scatter(stage.at[gh], [iota, cs], val)
                        pltpu.async_copy(stage.at[gh], out_hbm.at[b],
                                         sem2)
                        return gtot + 1

                    return lax.fori_loop(0, lax.div(scnt + L - 1, L),
                                         grp_body, carry1)

                return lax.fori_loop(0, nwin, win_body, carry0)

            gtot = lax.fori_loop(0, nrounds, round_body, 0)

            @pl.when(gtot >= 1)
            def _():
                pltpu.make_async_copy(
                    out_hbm.at[pl.ds(0, L)],
                    stage.at[lax.rem(gtot + 1, 2)], sem2).wait()

        pass64(uid_hbm, utT_hbm, tlu_hbm, u_out, N_USERS)
        pass64(iid_hbm, itT_hbm, tli_hbm, i_out, N_ITEMS)

        for c in range(NCHF):
            off = base + c * CHF
            pltpu.sync_copy(iid_hbm.at[pl.ds(off, CHF)], fidx)
            gv = pltpu.async_copy(vf_hbm.at[fidx], vbuf, sem)
            gt = pltpu.async_copy(tf_hbm.at[fidx], tbuf, sem)
            gv.wait()
            gt.wait()
            pltpu.sync_copy(vbuf, v_out.at[pl.ds(off, CHF)])
            pltpu.sync_copy(tbuf, t_out.at[pl.ds(off, CHF)])

    return _sc_gather


BLK = 1024


def _tc_body(u_ref, i_ref, v_ref, t_ref, wv_ref, wt_ref, wf_ref, bf_ref,
             out_ref):
    wf = wf_ref[...]
    a_vis = lax.dot_general(wf[:, :D_EMB], wv_ref[...],
                            (((1,), (0,)), ((), ())),
                            preferred_element_type=jnp.float32)
    a_txt = lax.dot_general(wf[:, D_EMB:], wt_ref[...],
                            (((1,), (0,)), ((), ())),
                            preferred_element_type=jnp.float32)
    mm = lax.dot_general(v_ref[...], a_vis, (((1,), (1,)), ((), ())),
                         preferred_element_type=jnp.float32)
    mm = mm + lax.dot_general(t_ref[...], a_txt, (((1,), (1,)), ((), ())),
                              preferred_element_type=jnp.float32)
    mm = mm + bf_ref[...]
    u = u_ref[...][:, :D_EMB]
    i = i_ref[...][:, :D_EMB]
    out_ref[...] = jnp.sum(u * (i + mm), axis=1)


_tc_score = pl.pallas_call(
    _tc_body,
    grid=(BATCH // BLK,),
    in_specs=[
        pl.BlockSpec((BLK, D_FEAT), lambda i: (i, 0)),
        pl.BlockSpec((BLK, D_FEAT), lambda i: (i, 0)),
        pl.BlockSpec((BLK, D_FEAT), lambda i: (i, 0)),
        pl.BlockSpec((BLK, D_FEAT), lambda i: (i, 0)),
        pl.BlockSpec((D_EMB, D_FEAT), lambda i: (0, 0)),
        pl.BlockSpec((D_EMB, D_FEAT), lambda i: (0, 0)),
        pl.BlockSpec((D_EMB, 2 * D_EMB), lambda i: (0, 0)),
        pl.BlockSpec((1, D_EMB), lambda i: (0, 0)),
    ],
    out_specs=pl.BlockSpec((BLK,), lambda i: (i,)),
    out_shape=jax.ShapeDtypeStruct((BATCH,), jnp.float32),
)


def kernel(user_ids, item_ids, user_table, item_table, visual_features,
           text_features, W_vis, W_txt, W_fuse, b_fuse):
    uid = user_ids.astype(jnp.int32)
    iid = item_ids.astype(jnp.int32)
    tlu = user_table[(N_USERS // 128) * 128:]
    tli = item_table[(N_ITEMS // 128) * 128:]
    u_g, i_g, v_g, t_g = _build_sc_gather()(uid, iid, user_table.T,
                                            item_table.T, tlu, tli,
                                            visual_features, text_features)
    return _tc_score(u_g, i_g, v_g, t_g, W_vis, W_txt, W_fuse,
                     b_fuse.reshape(1, D_EMB))
